# baseline (speedup 1.0000x reference)
import functools

import jax
import jax.numpy as jnp
from jax import lax
from jax.experimental import pallas as pl
from jax.experimental.pallas import tpu as pltpu
from jax.experimental.pallas import tpu_sc as plsc

NUM_CORES = 2
NUM_SUBCORES = 16
NW = NUM_CORES * NUM_SUBCORES

EMBED_DIM = 64
K = 26
C = 128


def _bag_sum_sc(idx3, table, n_bags, n_chunks):
    full_rounds = n_chunks // NW
    tail_workers = n_chunks - full_rounds * NW
    mesh = plsc.VectorSubcoreMesh(
        core_axis_name="c", subcore_axis_name="s",
        num_cores=NUM_CORES, num_subcores=NUM_SUBCORES)

    @functools.partial(
        pl.kernel,
        out_type=jax.ShapeDtypeStruct((n_bags, EMBED_DIM), jnp.float32),
        mesh=mesh,
        scratch_types=[
            pltpu.VMEM((2, K, C), jnp.int32),
            pltpu.VMEM((2, C, EMBED_DIM), jnp.float32),
            pltpu.SemaphoreType.DMA,
            pltpu.SemaphoreType.DMA,
        ],
        compiler_params=pltpu.CompilerParams(use_tc_tiling_on_sc=False, needs_layout_passes=False, skip_device_barrier=True),
    )
    def k(idx_hbm, table_hbm, out_hbm, idx_v, acc_v, sem0, sem1):
        wid = lax.axis_index("s") * NUM_CORES + lax.axis_index("c")
        sems = (sem0, sem1)
        zeros = jnp.zeros((16,), jnp.float32)

        def fire(buf, g):
            @pl.loop(0, C)
            def _z(r):
                for j in range(EMBED_DIM // 16):
                    acc_v[buf, r, pl.ds(16 * j, 16)] = zeros
            pltpu.sync_copy(idx_hbm.at[g], idx_v.at[buf])
            for j in range(K):
                pltpu.async_copy(table_hbm.at[idx_v.at[buf, j]],
                                 acc_v.at[buf], sems[buf], add=True)

        def drain(buf, g):
            cp = pltpu.make_async_copy(table_hbm.at[idx_v.at[buf, 0]],
                                       acc_v.at[buf], sems[buf])
            for _ in range(K):
                cp.wait()
            pltpu.sync_copy(acc_v.at[buf], out_hbm.at[pl.ds(g * C, C)])

        fire(0, wid)

        @pl.loop(0, (full_rounds - 2) // 2)
        def _pairs(u):
            g = wid + NW * (2 * u + 1)
            fire(1, g)
            drain(0, g - NW)
            fire(0, g + NW)
            drain(1, g)

        glast = wid + NW * (full_rounds - 1)
        fire(1, glast)
        drain(0, glast - NW)

        @pl.when(wid < tail_workers)
        def _tail_fire():
            fire(0, NW * full_rounds + wid)

        drain(1, glast)

        @pl.when(wid < tail_workers)
        def _tail_drain():
            drain(0, NW * full_rounds + wid)

    return k(idx3, table)


def kernel(x, table):
    batch, seq, k = x.shape
    n_bags = batch * seq
    n_chunks = n_bags // C
    idx = x.reshape(n_bags, k).astype(jnp.int32)
    idx3 = idx.reshape(n_chunks, C, k).transpose(0, 2, 1)
    out = _bag_sum_sc(idx3, table, n_bags, n_chunks)
    return out.reshape(batch, seq, EMBED_DIM)
